# baseline (speedup 1.0000x reference)
import numpy as np
import jax
import jax.numpy as jnp
from jax import lax
from jax.experimental import pallas as pl
from jax.experimental.pallas import tpu as pltpu

N_OBJ, N_DIM, N_SETS, N_NODE, B = 16, 1024, 8, 1024, 4096
BLK = 512
G = B // BLK + N_SETS
GB = G * BLK


def _mlp_body(meta_ref, x_ref, w1_ref, b1_ref, w2_ref, b2_ref, wh_ref,
              bh_ref, o_ref):
    gid = pl.program_id(0)

    @pl.when(gid < meta_ref[G])
    def _():
        x = x_ref[...]
        h = jnp.dot(x, w1_ref[...], preferred_element_type=jnp.float32)
        h = jnp.maximum(h + b1_ref[...], 0.0)
        h = jnp.dot(h.astype(jnp.bfloat16), w2_ref[...],
                    preferred_element_type=jnp.float32)
        h = jnp.maximum(h + b2_ref[...], 0.0)
        o = lax.dot_general(h.astype(jnp.bfloat16), wh_ref[0],
                            (((1,), (1,)), ((), ())),
                            preferred_element_type=jnp.float32)
        o_ref[...] = jax.nn.sigmoid(o + bh_ref[0])


_I0 = np.int32(0)

_grid_spec = pltpu.PrefetchScalarGridSpec(
    num_scalar_prefetch=1,
    grid=(G,),
    in_specs=[
        pl.BlockSpec((BLK, N_OBJ), lambda g, meta: (g, _I0)),
        pl.BlockSpec((N_OBJ, N_NODE), lambda g, meta: (_I0, _I0)),
        pl.BlockSpec((1, N_NODE), lambda g, meta: (_I0, _I0)),
        pl.BlockSpec((N_NODE, N_NODE), lambda g, meta: (_I0, _I0)),
        pl.BlockSpec((1, N_NODE), lambda g, meta: (_I0, _I0)),
        pl.BlockSpec((1, N_DIM, N_NODE), lambda g, meta: (meta[g], _I0, _I0)),
        pl.BlockSpec((1, 1, N_DIM), lambda g, meta: (meta[g], _I0, _I0)),
    ],
    out_specs=pl.BlockSpec((BLK, N_DIM), lambda g, meta: (g, _I0)),
)

_mlp_call = pl.pallas_call(
    _mlp_body,
    grid_spec=_grid_spec,
    out_shape=jax.ShapeDtypeStruct((GB, N_DIM), jnp.float32),
    compiler_params=pltpu.CompilerParams(
        dimension_semantics=("arbitrary",)),
)


@jax.jit
def kernel(pref, ps_id, W1, b1, W2, b2, Wh, bh):
    ps = ps_id.astype(jnp.int32)
    pref = pref.astype(jnp.float32)

    onehot = (ps[:, None] == jnp.arange(N_SETS, dtype=jnp.int32)[None, :]
              ).astype(jnp.int32)
    csum = jnp.cumsum(onehot, axis=0)
    counts = csum[-1]
    rank = jnp.sum(csum * onehot, axis=1) - 1

    nblk = (counts + BLK - 1) // BLK
    blk_cum0 = jnp.concatenate([jnp.zeros((1,), nblk.dtype),
                                jnp.cumsum(nblk)])
    my_base = jnp.sum(onehot * blk_cum0[None, :N_SETS], axis=1)
    slot = ((my_base + rank // BLK) * BLK + rank % BLK
            ).astype(jnp.int32)

    g = jnp.arange(G)
    bset = (jnp.sum(g[:, None] >= blk_cum0[None, 1:], axis=1)
            ).clip(0, N_SETS - 1)
    meta = jnp.concatenate([bset, blk_cum0[-1:]]).astype(jnp.int32)

    pref_sorted = jnp.zeros((GB, N_OBJ), jnp.float32).at[slot].set(pref)

    out_sorted = _mlp_call(
        meta, pref_sorted,
        W1.T.astype(jnp.float32),
        b1.reshape(1, N_NODE).astype(jnp.float32),
        W2.T.astype(jnp.bfloat16),
        b2.reshape(1, N_NODE).astype(jnp.float32),
        Wh.astype(jnp.bfloat16),
        bh.reshape(N_SETS, 1, N_DIM).astype(jnp.float32),
    )

    return out_sorted[slot].astype(jnp.float64)


# speedup vs baseline: 1.0243x; 1.0243x over previous
import functools

import numpy as np
import jax
import jax.numpy as jnp
from jax import lax
from jax.experimental import pallas as pl
from jax.experimental.pallas import tpu as pltpu
from jax.experimental.pallas import tpu_sc as plsc

N_OBJ, N_DIM, N_SETS, N_NODE, B = 16, 1024, 8, 1024, 4096
BLK = 512
G = B // BLK + N_SETS
GB = G * BLK


def _mlp_body(meta_ref, x_ref, w1_ref, b1_ref, w2_ref, b2_ref, wh_ref,
              bh_ref, o_ref):
    gid = pl.program_id(0)

    @pl.when(gid < meta_ref[G])
    def _():
        x = x_ref[...]
        h = jnp.dot(x, w1_ref[...], preferred_element_type=jnp.float32)
        h = jnp.maximum(h + b1_ref[...], 0.0)
        h = jnp.dot(h, w2_ref[...], preferred_element_type=jnp.float32)
        h = jnp.maximum(h + b2_ref[...], 0.0)
        o = lax.dot_general(h, wh_ref[0], (((1,), (1,)), ((), ())),
                            preferred_element_type=jnp.float32)
        o_ref[...] = jax.nn.sigmoid(o + bh_ref[0])


_SC_INFO = plsc.get_sparse_core_info()
_NW = _SC_INFO.num_cores * _SC_INFO.num_subcores
_RPW = B // _NW
_CH = 16
_NCHUNK = _RPW // _CH


def _unsort_body(table_hbm, idx_hbm, out_hbm, idx_v, rows_v, sem):
    wid = lax.axis_index("s") * _SC_INFO.num_cores + lax.axis_index("c")
    base = wid * _RPW
    for c in range(_NCHUNK):
        pltpu.sync_copy(idx_hbm.at[pl.ds(base + c * _CH, _CH)], idx_v)
        pltpu.async_copy(table_hbm.at[idx_v], rows_v, sem).wait()
        pltpu.sync_copy(rows_v, out_hbm.at[pl.ds(base + c * _CH, _CH)])


_unsort_call = functools.partial(
    pl.kernel,
    mesh=plsc.VectorSubcoreMesh(core_axis_name="c", subcore_axis_name="s"),
    out_type=jax.ShapeDtypeStruct((B, N_DIM), jnp.float32),
    scratch_types=[
        pltpu.VMEM((_CH,), jnp.int32),
        pltpu.VMEM((_CH, N_DIM), jnp.float32),
        pltpu.SemaphoreType.DMA,
    ],
)(_unsort_body)


_I0 = np.int32(0)

_grid_spec = pltpu.PrefetchScalarGridSpec(
    num_scalar_prefetch=1,
    grid=(G,),
    in_specs=[
        pl.BlockSpec((BLK, N_OBJ), lambda g, meta: (g, _I0)),
        pl.BlockSpec((N_OBJ, N_NODE), lambda g, meta: (_I0, _I0)),
        pl.BlockSpec((1, N_NODE), lambda g, meta: (_I0, _I0)),
        pl.BlockSpec((N_NODE, N_NODE), lambda g, meta: (_I0, _I0)),
        pl.BlockSpec((1, N_NODE), lambda g, meta: (_I0, _I0)),
        pl.BlockSpec((1, N_DIM, N_NODE), lambda g, meta: (meta[g], _I0, _I0)),
        pl.BlockSpec((1, 1, N_DIM), lambda g, meta: (meta[g], _I0, _I0)),
    ],
    out_specs=pl.BlockSpec((BLK, N_DIM), lambda g, meta: (g, _I0)),
)

_mlp_call = pl.pallas_call(
    _mlp_body,
    grid_spec=_grid_spec,
    out_shape=jax.ShapeDtypeStruct((GB, N_DIM), jnp.float32),
    compiler_params=pltpu.CompilerParams(
        dimension_semantics=("arbitrary",)),
)


@jax.jit
def kernel(pref, ps_id, W1, b1, W2, b2, Wh, bh):
    ps = ps_id.astype(jnp.int32)
    pref = pref.astype(jnp.float32)

    onehot = (ps[:, None] == jnp.arange(N_SETS, dtype=jnp.int32)[None, :]
              ).astype(jnp.int32)
    csum = jnp.cumsum(onehot, axis=0)
    counts = csum[-1]
    rank = jnp.sum(csum * onehot, axis=1) - 1

    nblk = (counts + BLK - 1) // BLK
    blk_cum0 = jnp.concatenate([jnp.zeros((1,), nblk.dtype),
                                jnp.cumsum(nblk)])
    my_base = jnp.sum(onehot * blk_cum0[None, :N_SETS], axis=1)
    slot = ((my_base + rank // BLK) * BLK + rank % BLK
            ).astype(jnp.int32)

    g = jnp.arange(G)
    bset = (jnp.sum(g[:, None] >= blk_cum0[None, 1:], axis=1)
            ).clip(0, N_SETS - 1)
    meta = jnp.concatenate([bset, blk_cum0[-1:]]).astype(jnp.int32)

    pref_sorted = jnp.zeros((GB, N_OBJ), jnp.float32).at[slot].set(pref)

    out_sorted = _mlp_call(
        meta, pref_sorted,
        W1.T.astype(jnp.float32),
        b1.reshape(1, N_NODE).astype(jnp.float32),
        W2.T.astype(jnp.float32),
        b2.reshape(1, N_NODE).astype(jnp.float32),
        Wh.astype(jnp.float32),
        bh.reshape(N_SETS, 1, N_DIM).astype(jnp.float32),
    )

    return _unsort_call(out_sorted, slot).astype(jnp.float64)


# speedup vs baseline: 1.0420x; 1.0173x over previous
import functools

import numpy as np
import jax
import jax.numpy as jnp
from jax import lax
from jax.experimental import pallas as pl
from jax.experimental.pallas import tpu as pltpu
from jax.experimental.pallas import tpu_sc as plsc

N_OBJ, N_DIM, N_SETS, N_NODE, B = 16, 1024, 8, 1024, 4096
BLK = 512
G = B // BLK + N_SETS
GB = G * BLK


def _mlp_body(meta_ref, x_ref, w1_ref, b1_ref, w2_ref, b2_ref, wh_ref,
              bh_ref, o_ref):
    gid = pl.program_id(0)

    @pl.when(gid < meta_ref[G])
    def _():
        x = x_ref[...]
        h = jnp.dot(x, w1_ref[...], preferred_element_type=jnp.float32)
        h = jnp.maximum(h + b1_ref[...], 0.0)
        h = jnp.dot(h, w2_ref[...], preferred_element_type=jnp.float32)
        h = jnp.maximum(h + b2_ref[...], 0.0)
        o = lax.dot_general(h, wh_ref[0], (((1,), (1,)), ((), ())),
                            preferred_element_type=jnp.float32)
        o_ref[...] = jax.nn.sigmoid(o + bh_ref[0])


_SC_INFO = plsc.get_sparse_core_info()
_NW = _SC_INFO.num_cores * _SC_INFO.num_subcores
_RPW = B // _NW
_CH = 16
_NCHUNK = _RPW // _CH


def _unsort_body(table_hbm, idx_hbm, out_hbm, idx_v, rows_v0, rows_v1,
                 sem0, sem1):
    wid = lax.axis_index("s") * _SC_INFO.num_cores + lax.axis_index("c")
    base = wid * _RPW
    pltpu.sync_copy(idx_hbm.at[pl.ds(base, _RPW)], idx_v)
    bufs = (rows_v0, rows_v1)
    sems = (sem0, sem1)
    copies = []
    for c in range(_NCHUNK):
        copies.append(pltpu.async_copy(
            table_hbm.at[idx_v.at[pl.ds(c * _CH, _CH)]], bufs[c % 2],
            sems[c % 2]))
        if c >= 1:
            copies[c - 1].wait()
            pltpu.sync_copy(bufs[(c - 1) % 2],
                            out_hbm.at[pl.ds(base + (c - 1) * _CH, _CH)])
    copies[_NCHUNK - 1].wait()
    pltpu.sync_copy(bufs[(_NCHUNK - 1) % 2],
                    out_hbm.at[pl.ds(base + (_NCHUNK - 1) * _CH, _CH)])


_unsort_call = functools.partial(
    pl.kernel,
    mesh=plsc.VectorSubcoreMesh(core_axis_name="c", subcore_axis_name="s"),
    out_type=jax.ShapeDtypeStruct((B, N_DIM), jnp.float32),
    scratch_types=[
        pltpu.VMEM((_RPW,), jnp.int32),
        pltpu.VMEM((_CH, N_DIM), jnp.float32),
        pltpu.VMEM((_CH, N_DIM), jnp.float32),
        pltpu.SemaphoreType.DMA,
        pltpu.SemaphoreType.DMA,
    ],
)(_unsort_body)


_I0 = np.int32(0)

_grid_spec = pltpu.PrefetchScalarGridSpec(
    num_scalar_prefetch=1,
    grid=(G,),
    in_specs=[
        pl.BlockSpec((BLK, N_OBJ), lambda g, meta: (g, _I0)),
        pl.BlockSpec((N_OBJ, N_NODE), lambda g, meta: (_I0, _I0)),
        pl.BlockSpec((1, N_NODE), lambda g, meta: (_I0, _I0)),
        pl.BlockSpec((N_NODE, N_NODE), lambda g, meta: (_I0, _I0)),
        pl.BlockSpec((1, N_NODE), lambda g, meta: (_I0, _I0)),
        pl.BlockSpec((1, N_DIM, N_NODE), lambda g, meta: (meta[g], _I0, _I0)),
        pl.BlockSpec((1, 1, N_DIM), lambda g, meta: (meta[g], _I0, _I0)),
    ],
    out_specs=pl.BlockSpec((BLK, N_DIM), lambda g, meta: (g, _I0)),
)

_mlp_call = pl.pallas_call(
    _mlp_body,
    grid_spec=_grid_spec,
    out_shape=jax.ShapeDtypeStruct((GB, N_DIM), jnp.float32),
    compiler_params=pltpu.CompilerParams(
        dimension_semantics=("arbitrary",)),
)


@jax.jit
def kernel(pref, ps_id, W1, b1, W2, b2, Wh, bh):
    ps = ps_id.astype(jnp.int32)
    pref = pref.astype(jnp.float32)

    onehot = (ps[:, None] == jnp.arange(N_SETS, dtype=jnp.int32)[None, :]
              ).astype(jnp.int32)
    csum = jnp.cumsum(onehot, axis=0)
    counts = csum[-1]
    rank = jnp.sum(csum * onehot, axis=1) - 1

    nblk = (counts + BLK - 1) // BLK
    blk_cum0 = jnp.concatenate([jnp.zeros((1,), nblk.dtype),
                                jnp.cumsum(nblk)])
    my_base = jnp.sum(onehot * blk_cum0[None, :N_SETS], axis=1)
    slot = ((my_base + rank // BLK) * BLK + rank % BLK
            ).astype(jnp.int32)

    g = jnp.arange(G)
    bset = (jnp.sum(g[:, None] >= blk_cum0[None, 1:], axis=1)
            ).clip(0, N_SETS - 1)
    meta = jnp.concatenate([bset, blk_cum0[-1:]]).astype(jnp.int32)

    pref_sorted = jnp.zeros((GB, N_OBJ), jnp.float32).at[slot].set(pref)

    out_sorted = _mlp_call(
        meta, pref_sorted,
        W1.T.astype(jnp.float32),
        b1.reshape(1, N_NODE).astype(jnp.float32),
        W2.T.astype(jnp.float32),
        b2.reshape(1, N_NODE).astype(jnp.float32),
        Wh.astype(jnp.float32),
        bh.reshape(N_SETS, 1, N_DIM).astype(jnp.float32),
    )

    return _unsort_call(out_sorted, slot).astype(jnp.float64)
